# baseline (speedup 1.0000x reference)
import functools
import math

import jax
import jax.numpy as jnp
from jax import lax
from jax.experimental import pallas as pl
from jax.experimental.pallas import tpu as pltpu
from jax.experimental.pallas import tpu_sc as plsc

VOCAB_SIZE = 5561
COORD_TOKEN_BASE = 100
D_MODEL = 256
D_TYPE = 32
D_COORD = 64
N_TYPES = 7
SEQ = 200

VPAD = 5568
NC, NS = 2, 16
NW = NC * NS
CH = 64
L = 16
NVEC = D_MODEL // L


def _gelu_exact(x):
    return x * 0.5 * (1.0 + lax.erf(x * (1.0 / math.sqrt(2.0))))


def _fused_table_body(tt, tt8, tidl, cl, w1, b1r, w2, wp_tok, wp_typ, wp_c,
                      b2r, bpr, out):
    acc = jnp.dot(tt[:], wp_tok[:], preferred_element_type=jnp.float32)
    tp = jnp.dot(tt8[:], wp_typ[:], preferred_element_type=jnp.float32)
    oh = (tidl[:] == lax.broadcasted_iota(jnp.int32, (VPAD, 8), 1))
    acc += jnp.dot(oh.astype(jnp.float32), tp,
                   preferred_element_type=jnp.float32)
    rows = lax.broadcasted_iota(jnp.int32, (VPAD, 8), 0)
    clm = jnp.where(rows >= COORD_TOKEN_BASE, cl[:], 0.0)
    h = _gelu_exact(jnp.dot(clm, w1[:], preferred_element_type=jnp.float32)
                    + b1r[:])
    w2c = jnp.dot(w2[:], wp_c[:], preferred_element_type=jnp.float32)
    acc += jnp.dot(h, w2c, preferred_element_type=jnp.float32)
    acc += jnp.dot(b2r[:], wp_c[:], preferred_element_type=jnp.float32)
    acc += bpr[:]
    out[:] = acc


def _rsqrt_newton(v):
    i = lax.bitcast_convert_type(v, jnp.int32)
    i = jnp.int32(0x5F3759DF) - (i >> 1)
    r = lax.bitcast_convert_type(i, jnp.float32)
    for _ in range(3):
        r = r * (1.5 - 0.5 * v * r * r)
    return r


def _lane_sum(x):
    idx = lax.iota(jnp.int32, L)
    for k in (8, 4, 2, 1):
        x = x + x.at[idx ^ k].get(mode="promise_in_bounds")
    return x


def _sc_lookup_ln(n_chunks, n_tok):
    mesh = plsc.VectorSubcoreMesh(core_axis_name="c", subcore_axis_name="s")
    pos_per_chunk = CH // 32

    @functools.partial(
        pl.kernel,
        out_type=jax.ShapeDtypeStruct((n_tok, D_MODEL), jnp.float32),
        mesh=mesh,
        scratch_types=[
            pltpu.VMEM((n_chunks, CH), jnp.int32),
            pltpu.VMEM((n_chunks, CH), jnp.int32),
            pltpu.VMEM((CH, D_MODEL), jnp.float32),
            pltpu.VMEM((CH, D_MODEL), jnp.float32),
            pltpu.VMEM((CH, D_MODEL), jnp.float32),
            pltpu.VMEM((CH, D_MODEL), jnp.float32),
            pltpu.VMEM((2, pos_per_chunk, D_MODEL), jnp.float32),
            pltpu.VMEM((D_MODEL,), jnp.float32),
            pltpu.VMEM((D_MODEL,), jnp.float32),
            pltpu.SemaphoreType.DMA,
            pltpu.SemaphoreType.DMA,
            pltpu.SemaphoreType.DMA,
            pltpu.SemaphoreType.DMA,
            pltpu.SemaphoreType.DMA,
            pltpu.SemaphoreType.DMA,
        ],
    )
    def body(gidx_hbm, sidx_hbm, table_hbm, pe_hbm, gamma_hbm, beta_hbm,
             out_hbm, gidx_v, sidx_v, g0, g1, o0, o1, pe_v, gam_v, bet_v,
             gsem0, gsem1, ssem0, ssem1, psem0, psem1):
        wid = lax.axis_index("s") * NC + lax.axis_index("c")
        pltpu.sync_copy(gidx_hbm.at[wid], gidx_v)
        pltpu.sync_copy(sidx_hbm.at[wid], sidx_v)
        pltpu.sync_copy(gamma_hbm, gam_v)
        pltpu.sync_copy(beta_hbm, bet_v)

        gbufs = (g0, g1)
        obufs = (o0, o1)
        gsems = (gsem0, gsem1)
        ssems = (ssem0, ssem1)
        psems = (psem0, psem1)

        def gather_desc(c, b):
            return pltpu.make_async_copy(table_hbm.at[gidx_v.at[c]],
                                         gbufs[b], gsems[b])

        def scatter_desc(c, b):
            return pltpu.make_async_copy(obufs[b], out_hbm.at[sidx_v.at[c]],
                                         ssems[b])

        def pe_desc(c, b):
            return pltpu.make_async_copy(
                pe_hbm.at[pl.ds(c * pos_per_chunk, pos_per_chunk)],
                pe_v.at[b], psems[b])

        def start_chunk(c, b):
            gather_desc(c, b).start()
            pe_desc(c, b).start()

        gam = [gam_v[pl.ds(j * L, L)] for j in range(NVEC)]
        bet = [bet_v[pl.ds(j * L, L)] for j in range(NVEC)]

        def process_chunk(c, b):
            gbuf, obuf = gbufs[b], obufs[b]
            for h in range(pos_per_chunk):
                pe_h = [pe_v[b, h, pl.ds(j * L, L)] for j in range(NVEC)]

                def do_token(t):
                    s = jnp.zeros((L,), jnp.float32)
                    q = jnp.zeros((L,), jnp.float32)
                    for j in range(NVEC):
                        x = gbuf[t, pl.ds(j * L, L)] + pe_h[j]
                        s = s + x
                        q = q + x * x
                    mean = _lane_sum(s) * (1.0 / D_MODEL)
                    var = _lane_sum(q) * (1.0 / D_MODEL) - mean * mean
                    rstd = _rsqrt_newton(var + 1e-5)
                    shift = -mean * rstd
                    for j in range(NVEC):
                        x = gbuf[t, pl.ds(j * L, L)] + pe_h[j]
                        y = x * rstd + shift
                        obuf[t, pl.ds(j * L, L)] = y * gam[j] + bet[j]

                @plsc.parallel_loop(h * 32, h * 32 + 32, step=1, unroll=3)
                def _(t):
                    do_token(t)

        start_chunk(0, 0)
        start_chunk(1, 1)

        def chunk_pair(cc, _):
            for b in range(2):
                c = cc * 2 + b

                @pl.when(c < n_chunks)
                def _():
                    gather_desc(c, b).wait()
                    pe_desc(c, b).wait()

                    @pl.when(c >= 2)
                    def _():
                        scatter_desc(c - 2, b).wait()

                    process_chunk(c, b)
                    scatter_desc(c, b).start()

                    @pl.when(c + 2 < n_chunks)
                    def _():
                        start_chunk(c + 2, b)
            return 0

        lax.fori_loop(0, (n_chunks + 1) // 2, chunk_pair, 0)
        scatter_desc(n_chunks - 2, (n_chunks - 2) % 2).wait()
        scatter_desc(n_chunks - 1, (n_chunks - 1) % 2).wait()

    return body


def kernel(token_ids, token_table, type_table, W1, b1, W2, b2, Wp, bp,
           gamma, beta, coord_lut, pe, type_id_lut):
    B, S = token_ids.shape
    n_tok = B * S
    per_w = n_tok // NW
    n_chunks = per_w // CH

    tt = jnp.zeros((VPAD, D_MODEL), jnp.float32).at[:VOCAB_SIZE].set(token_table)
    tt8 = jnp.zeros((8, D_TYPE), jnp.float32).at[:N_TYPES].set(type_table)
    tidl = jnp.zeros((VPAD, 1), jnp.int32).at[:VOCAB_SIZE, 0].set(
        type_id_lut.astype(jnp.int32))
    cl = jnp.zeros((VPAD, 8), jnp.float32).at[:VOCAB_SIZE, :3].set(coord_lut)
    w1 = jnp.zeros((8, D_COORD), jnp.float32).at[:3].set(W1)
    fused = pl.pallas_call(
        _fused_table_body,
        out_shape=jax.ShapeDtypeStruct((VPAD, D_MODEL), jnp.float32),
    )(tt, tt8, tidl, cl, w1, b1.reshape(1, D_COORD), W2,
      Wp[:D_MODEL], Wp[D_MODEL:D_MODEL + D_TYPE], Wp[D_MODEL + D_TYPE:],
      b2.reshape(1, D_COORD), bp.reshape(1, D_MODEL))

    seq_per_w = B // NW
    ids = jnp.clip(token_ids.astype(jnp.int32), 0, VOCAB_SIZE - 1)
    gidx = (ids.reshape(NW, seq_per_w, S)
            .transpose(0, 2, 1)
            .reshape(NW, n_chunks, CH))
    w_ = jnp.arange(NW, dtype=jnp.int32).reshape(NW, 1, 1, 1)
    c_ = jnp.arange(n_chunks, dtype=jnp.int32).reshape(1, n_chunks, 1, 1)
    h_ = jnp.arange(CH // seq_per_w, dtype=jnp.int32).reshape(1, 1, -1, 1)
    s_ = jnp.arange(seq_per_w, dtype=jnp.int32).reshape(1, 1, 1, seq_per_w)
    sidx = ((w_ * seq_per_w + s_) * S + (CH // seq_per_w) * c_ + h_)
    sidx = sidx.reshape(NW, n_chunks, CH)
    out = _sc_lookup_ln(n_chunks, n_tok)(gidx, sidx, fused, pe[:SEQ],
                                         gamma, beta)
    return out.reshape(B, S, D_MODEL)


# speedup vs baseline: 1.1906x; 1.1906x over previous
import functools
import math

import jax
import jax.numpy as jnp
from jax import lax
from jax.experimental import pallas as pl
from jax.experimental.pallas import tpu as pltpu
from jax.experimental.pallas import tpu_sc as plsc

VOCAB_SIZE = 5561
COORD_TOKEN_BASE = 100
D_MODEL = 256
D_TYPE = 32
D_COORD = 64
N_TYPES = 7
SEQ = 200

VPAD = 5568
NC, NS = 2, 16
NW = NC * NS
CH = 64
L = 16
NVEC = D_MODEL // L


def _gelu_exact(x):
    return x * 0.5 * (1.0 + lax.erf(x * (1.0 / math.sqrt(2.0))))


def _fused_table_body(tt, tt8, tidl, cl, w1, b1r, w2, wp_tok, wp_typ, wp_c,
                      b2r, bpr, out):
    acc = jnp.dot(tt[:], wp_tok[:], preferred_element_type=jnp.float32)
    tp = jnp.dot(tt8[:], wp_typ[:], preferred_element_type=jnp.float32)
    oh = (tidl[:] == lax.broadcasted_iota(jnp.int32, (VPAD, 8), 1))
    acc += jnp.dot(oh.astype(jnp.float32), tp,
                   preferred_element_type=jnp.float32)
    rows = lax.broadcasted_iota(jnp.int32, (VPAD, 8), 0)
    clm = jnp.where(rows >= COORD_TOKEN_BASE, cl[:], 0.0)
    h = _gelu_exact(jnp.dot(clm, w1[:], preferred_element_type=jnp.float32)
                    + b1r[:])
    w2c = jnp.dot(w2[:], wp_c[:], preferred_element_type=jnp.float32)
    acc += jnp.dot(h, w2c, preferred_element_type=jnp.float32)
    acc += jnp.dot(b2r[:], wp_c[:], preferred_element_type=jnp.float32)
    acc += bpr[:]
    out[:] = acc


def _rsqrt_newton(v):
    i = lax.bitcast_convert_type(v, jnp.int32)
    i = jnp.int32(0x5F3759DF) - (i >> 1)
    r = lax.bitcast_convert_type(i, jnp.float32)
    for _ in range(3):
        r = r * (1.5 - 0.5 * v * r * r)
    return r


def _lane_sum(x):
    idx = lax.iota(jnp.int32, L)
    for k in (8, 4, 2, 1):
        x = x + x.at[idx ^ k].get(mode="promise_in_bounds")
    return x


def _sc_lookup_ln(n_chunks, n_tok):
    mesh = plsc.VectorSubcoreMesh(core_axis_name="c", subcore_axis_name="s")
    pos_per_chunk = CH // 32

    @functools.partial(
        pl.kernel,
        out_type=jax.ShapeDtypeStruct((n_tok, D_MODEL), jnp.float32),
        mesh=mesh,
        scratch_types=[
            pltpu.VMEM((n_chunks, CH), jnp.int32),
            pltpu.VMEM((n_chunks, CH), jnp.int32),
            pltpu.VMEM((CH, D_MODEL), jnp.float32),
            pltpu.VMEM((CH, D_MODEL), jnp.float32),
            pltpu.VMEM((CH, D_MODEL), jnp.float32),
            pltpu.VMEM((CH, D_MODEL), jnp.float32),
            pltpu.VMEM((2, pos_per_chunk, D_MODEL), jnp.float32),
            pltpu.VMEM((CH, 2, L), jnp.float32),
            pltpu.VMEM((D_MODEL,), jnp.float32),
            pltpu.VMEM((D_MODEL,), jnp.float32),
            pltpu.SemaphoreType.DMA,
            pltpu.SemaphoreType.DMA,
            pltpu.SemaphoreType.DMA,
            pltpu.SemaphoreType.DMA,
            pltpu.SemaphoreType.DMA,
            pltpu.SemaphoreType.DMA,
        ],
    )
    def body(gidx_hbm, sidx_hbm, table_hbm, pe_hbm, gamma_hbm, beta_hbm,
             out_hbm, gidx_v, sidx_v, g0, g1, o0, o1, pe_v, st_v, gam_v,
             bet_v, gsem0, gsem1, ssem0, ssem1, psem0, psem1):
        wid = lax.axis_index("s") * NC + lax.axis_index("c")
        pltpu.sync_copy(gidx_hbm.at[wid], gidx_v)
        pltpu.sync_copy(sidx_hbm.at[wid], sidx_v)
        pltpu.sync_copy(gamma_hbm, gam_v)
        pltpu.sync_copy(beta_hbm, bet_v)

        gbufs = (g0, g1)
        obufs = (o0, o1)
        gsems = (gsem0, gsem1)
        ssems = (ssem0, ssem1)
        psems = (psem0, psem1)

        def gather_desc(c, b):
            return pltpu.make_async_copy(table_hbm.at[gidx_v.at[c]],
                                         gbufs[b], gsems[b])

        def scatter_desc(c, b):
            return pltpu.make_async_copy(obufs[b], out_hbm.at[sidx_v.at[c]],
                                         ssems[b])

        def pe_desc(c, b):
            return pltpu.make_async_copy(
                pe_hbm.at[pl.ds(c * pos_per_chunk, pos_per_chunk)],
                pe_v.at[b], psems[b])

        def start_chunk(c, b):
            gather_desc(c, b).start()
            pe_desc(c, b).start()

        gam = [gam_v[pl.ds(j * L, L)] for j in range(NVEC)]
        bet = [bet_v[pl.ds(j * L, L)] for j in range(NVEC)]

        def process_chunk(c, b):
            gbuf, obuf = gbufs[b], obufs[b]
            for h in range(pos_per_chunk):
                pe_h = [pe_v[b, h, pl.ds(j * L, L)] for j in range(NVEC)]

                @plsc.parallel_loop(h * 32, h * 32 + 32, step=1, unroll=4)
                def _(t):
                    s = jnp.zeros((L,), jnp.float32)
                    q = jnp.zeros((L,), jnp.float32)
                    for j in range(NVEC):
                        x = gbuf[t, pl.ds(j * L, L)] + pe_h[j]
                        s = s + x
                        q = q + x * x
                    mean = _lane_sum(s) * (1.0 / D_MODEL)
                    var = _lane_sum(q) * (1.0 / D_MODEL) - mean * mean
                    rstd = _rsqrt_newton(var + 1e-5)
                    st_v[t, 0] = rstd
                    st_v[t, 1] = -mean * rstd

                @plsc.parallel_loop(h * 32, h * 32 + 32, step=1, unroll=2)
                def _(t):
                    rstd = st_v[t, 0]
                    shift = st_v[t, 1]
                    for j in range(NVEC):
                        x = gbuf[t, pl.ds(j * L, L)] + pe_h[j]
                        y = x * rstd + shift
                        obuf[t, pl.ds(j * L, L)] = y * gam[j] + bet[j]

        start_chunk(0, 0)
        start_chunk(1, 1)

        def chunk_pair(cc, _):
            for b in range(2):
                c = cc * 2 + b

                @pl.when(c < n_chunks)
                def _():
                    gather_desc(c, b).wait()
                    pe_desc(c, b).wait()

                    @pl.when(c >= 2)
                    def _():
                        scatter_desc(c - 2, b).wait()

                    process_chunk(c, b)
                    scatter_desc(c, b).start()

                    @pl.when(c + 2 < n_chunks)
                    def _():
                        start_chunk(c + 2, b)
            return 0

        lax.fori_loop(0, (n_chunks + 1) // 2, chunk_pair, 0)
        scatter_desc(n_chunks - 2, (n_chunks - 2) % 2).wait()
        scatter_desc(n_chunks - 1, (n_chunks - 1) % 2).wait()

    return body


def kernel(token_ids, token_table, type_table, W1, b1, W2, b2, Wp, bp,
           gamma, beta, coord_lut, pe, type_id_lut):
    B, S = token_ids.shape
    n_tok = B * S
    per_w = n_tok // NW
    n_chunks = per_w // CH

    tt = jnp.zeros((VPAD, D_MODEL), jnp.float32).at[:VOCAB_SIZE].set(token_table)
    tt8 = jnp.zeros((8, D_TYPE), jnp.float32).at[:N_TYPES].set(type_table)
    tidl = jnp.zeros((VPAD, 1), jnp.int32).at[:VOCAB_SIZE, 0].set(
        type_id_lut.astype(jnp.int32))
    cl = jnp.zeros((VPAD, 8), jnp.float32).at[:VOCAB_SIZE, :3].set(coord_lut)
    w1 = jnp.zeros((8, D_COORD), jnp.float32).at[:3].set(W1)
    fused = pl.pallas_call(
        _fused_table_body,
        out_shape=jax.ShapeDtypeStruct((VPAD, D_MODEL), jnp.float32),
    )(tt, tt8, tidl, cl, w1, b1.reshape(1, D_COORD), W2,
      Wp[:D_MODEL], Wp[D_MODEL:D_MODEL + D_TYPE], Wp[D_MODEL + D_TYPE:],
      b2.reshape(1, D_COORD), bp.reshape(1, D_MODEL))

    seq_per_w = B // NW
    ids = jnp.clip(token_ids.astype(jnp.int32), 0, VOCAB_SIZE - 1)
    gidx = (ids.reshape(NW, seq_per_w, S)
            .transpose(0, 2, 1)
            .reshape(NW, n_chunks, CH))
    w_ = jnp.arange(NW, dtype=jnp.int32).reshape(NW, 1, 1, 1)
    c_ = jnp.arange(n_chunks, dtype=jnp.int32).reshape(1, n_chunks, 1, 1)
    h_ = jnp.arange(CH // seq_per_w, dtype=jnp.int32).reshape(1, 1, -1, 1)
    s_ = jnp.arange(seq_per_w, dtype=jnp.int32).reshape(1, 1, 1, seq_per_w)
    sidx = ((w_ * seq_per_w + s_) * S + (CH // seq_per_w) * c_ + h_)
    sidx = sidx.reshape(NW, n_chunks, CH)
    out = _sc_lookup_ln(n_chunks, n_tok)(gidx, sidx, fused, pe[:SEQ],
                                         gamma, beta)
    return out.reshape(B, S, D_MODEL)


# speedup vs baseline: 1.3832x; 1.1617x over previous
import functools
import math

import jax
import jax.numpy as jnp
from jax import lax
from jax.experimental import pallas as pl
from jax.experimental.pallas import tpu as pltpu
from jax.experimental.pallas import tpu_sc as plsc

VOCAB_SIZE = 5561
COORD_TOKEN_BASE = 100
D_MODEL = 256
D_TYPE = 32
D_COORD = 64
N_TYPES = 7
SEQ = 200

VPAD = 5568
NC, NS = 2, 16
NW = NC * NS
CH = 64
L = 16
NVEC = D_MODEL // L


def _gelu_exact(x):
    return x * 0.5 * (1.0 + lax.erf(x * (1.0 / math.sqrt(2.0))))


def _fused_table_body(tt, tt8, tidl, cl, w1, b1r, w2, wp_tok, wp_typ, wp_c,
                      b2r, bpr, out):
    acc = jnp.dot(tt[:], wp_tok[:], preferred_element_type=jnp.float32)
    tp = jnp.dot(tt8[:], wp_typ[:], preferred_element_type=jnp.float32)
    oh = (tidl[:] == lax.broadcasted_iota(jnp.int32, (VPAD, 8), 1))
    acc += jnp.dot(oh.astype(jnp.float32), tp,
                   preferred_element_type=jnp.float32)
    rows = lax.broadcasted_iota(jnp.int32, (VPAD, 8), 0)
    clm = jnp.where(rows >= COORD_TOKEN_BASE, cl[:], 0.0)
    h = _gelu_exact(jnp.dot(clm, w1[:], preferred_element_type=jnp.float32)
                    + b1r[:])
    w2c = jnp.dot(w2[:], wp_c[:], preferred_element_type=jnp.float32)
    acc += jnp.dot(h, w2c, preferred_element_type=jnp.float32)
    acc += jnp.dot(b2r[:], wp_c[:], preferred_element_type=jnp.float32)
    acc += bpr[:]
    out[:] = acc


def _rsqrt_newton(v):
    i = lax.bitcast_convert_type(v, jnp.int32)
    i = jnp.int32(0x5F3759DF) - (i >> 1)
    r = lax.bitcast_convert_type(i, jnp.float32)
    for _ in range(3):
        r = r * (1.5 - 0.5 * v * r * r)
    return r


def _lane_sum(x):
    idx = lax.iota(jnp.int32, L)
    for k in (8, 4, 2, 1):
        x = x + x.at[idx ^ k].get(mode="promise_in_bounds")
    return x


def _sc_lookup_ln(n_chunks, n_tok):
    mesh = plsc.VectorSubcoreMesh(core_axis_name="c", subcore_axis_name="s")
    pos_per_chunk = CH // 32

    @functools.partial(
        pl.kernel,
        out_type=jax.ShapeDtypeStruct((n_tok, D_MODEL), jnp.float32),
        mesh=mesh,
        scratch_types=[
            pltpu.VMEM((n_chunks, CH), jnp.int32),
            pltpu.VMEM((n_chunks, CH), jnp.int32),
            pltpu.VMEM((CH, D_MODEL), jnp.float32),
            pltpu.VMEM((CH, D_MODEL), jnp.float32),
            pltpu.VMEM((CH, D_MODEL), jnp.float32),
            pltpu.VMEM((CH, D_MODEL), jnp.float32),
            pltpu.VMEM((2, pos_per_chunk, D_MODEL), jnp.float32),
            pltpu.VMEM((CH, 2, L), jnp.float32),
            pltpu.VMEM((D_MODEL,), jnp.float32),
            pltpu.VMEM((D_MODEL,), jnp.float32),
            pltpu.SemaphoreType.DMA,
            pltpu.SemaphoreType.DMA,
            pltpu.SemaphoreType.DMA,
            pltpu.SemaphoreType.DMA,
            pltpu.SemaphoreType.DMA,
            pltpu.SemaphoreType.DMA,
        ],
    )
    def body(gidx_hbm, sidx_hbm, table_hbm, pe_hbm, gamma_hbm, beta_hbm,
             out_hbm, gidx_v, sidx_v, g0, g1, o0, o1, pe_v, st_v, gam_v,
             bet_v, gsem0, gsem1, ssem0, ssem1, psem0, psem1):
        wid = lax.axis_index("s") * NC + lax.axis_index("c")
        pltpu.sync_copy(gidx_hbm.at[wid], gidx_v)
        pltpu.sync_copy(sidx_hbm.at[wid], sidx_v)
        pltpu.sync_copy(gamma_hbm, gam_v)
        pltpu.sync_copy(beta_hbm, bet_v)

        gbufs = (g0, g1)
        obufs = (o0, o1)
        gsems = (gsem0, gsem1)
        ssems = (ssem0, ssem1)
        psems = (psem0, psem1)

        def gather_desc(c, b):
            return pltpu.make_async_copy(table_hbm.at[gidx_v.at[c]],
                                         gbufs[b], gsems[b])

        def scatter_desc(c, b):
            return pltpu.make_async_copy(obufs[b], out_hbm.at[sidx_v.at[c]],
                                         ssems[b])

        def pe_desc(c, b):
            return pltpu.make_async_copy(
                pe_hbm.at[pl.ds(c * pos_per_chunk, pos_per_chunk)],
                pe_v.at[b], psems[b])

        def start_chunk(c, b):
            gather_desc(c, b).start()
            pe_desc(c, b).start()

        gam = [gam_v[pl.ds(j * L, L)] for j in range(NVEC)]
        bet = [bet_v[pl.ds(j * L, L)] for j in range(NVEC)]

        def process_chunk(c, b):
            gbuf, obuf = gbufs[b], obufs[b]
            for h in range(pos_per_chunk):
                pe_h = [pe_v[b, h, pl.ds(j * L, L)] for j in range(NVEC)]

                @plsc.parallel_loop(h * 32, h * 32 + 32, step=1, unroll=2)
                def _(t):
                    s = jnp.zeros((L,), jnp.float32)
                    q = jnp.zeros((L,), jnp.float32)
                    for j in range(NVEC):
                        x = gbuf[t, pl.ds(j * L, L)] + pe_h[j]
                        s = s + x
                        q = q + x * x
                    mean = _lane_sum(s) * (1.0 / D_MODEL)
                    var = _lane_sum(q) * (1.0 / D_MODEL) - mean * mean
                    rstd = _rsqrt_newton(var + 1e-5)
                    st_v[t, 0] = rstd
                    st_v[t, 1] = -mean * rstd

                @plsc.parallel_loop(h * 32, h * 32 + 32, step=1, unroll=2)
                def _(t):
                    rstd = st_v[t, 0]
                    shift = st_v[t, 1]
                    for j in range(NVEC):
                        x = gbuf[t, pl.ds(j * L, L)] + pe_h[j]
                        y = x * rstd + shift
                        obuf[t, pl.ds(j * L, L)] = y * gam[j] + bet[j]

        start_chunk(0, 0)
        start_chunk(1, 1)

        def chunk_pair(cc, _):
            for b in range(2):
                c = cc * 2 + b

                @pl.when(c < n_chunks)
                def _():
                    gather_desc(c, b).wait()
                    pe_desc(c, b).wait()

                    @pl.when(c >= 2)
                    def _():
                        scatter_desc(c - 2, b).wait()

                    process_chunk(c, b)
                    scatter_desc(c, b).start()

                    @pl.when(c + 2 < n_chunks)
                    def _():
                        start_chunk(c + 2, b)
            return 0

        lax.fori_loop(0, (n_chunks + 1) // 2, chunk_pair, 0)
        scatter_desc(n_chunks - 2, (n_chunks - 2) % 2).wait()
        scatter_desc(n_chunks - 1, (n_chunks - 1) % 2).wait()

    return body


def kernel(token_ids, token_table, type_table, W1, b1, W2, b2, Wp, bp,
           gamma, beta, coord_lut, pe, type_id_lut):
    B, S = token_ids.shape
    n_tok = B * S
    per_w = n_tok // NW
    n_chunks = per_w // CH

    tt = jnp.zeros((VPAD, D_MODEL), jnp.float32).at[:VOCAB_SIZE].set(token_table)
    tt8 = jnp.zeros((8, D_TYPE), jnp.float32).at[:N_TYPES].set(type_table)
    tidl = jnp.zeros((VPAD, 1), jnp.int32).at[:VOCAB_SIZE, 0].set(
        type_id_lut.astype(jnp.int32))
    cl = jnp.zeros((VPAD, 8), jnp.float32).at[:VOCAB_SIZE, :3].set(coord_lut)
    w1 = jnp.zeros((8, D_COORD), jnp.float32).at[:3].set(W1)
    fused = pl.pallas_call(
        _fused_table_body,
        out_shape=jax.ShapeDtypeStruct((VPAD, D_MODEL), jnp.float32),
    )(tt, tt8, tidl, cl, w1, b1.reshape(1, D_COORD), W2,
      Wp[:D_MODEL], Wp[D_MODEL:D_MODEL + D_TYPE], Wp[D_MODEL + D_TYPE:],
      b2.reshape(1, D_COORD), bp.reshape(1, D_MODEL))

    seq_per_w = B // NW
    ids = jnp.clip(token_ids.astype(jnp.int32), 0, VOCAB_SIZE - 1)
    gidx = (ids.reshape(NW, seq_per_w, S)
            .transpose(0, 2, 1)
            .reshape(NW, n_chunks, CH))
    w_ = jnp.arange(NW, dtype=jnp.int32).reshape(NW, 1, 1, 1)
    c_ = jnp.arange(n_chunks, dtype=jnp.int32).reshape(1, n_chunks, 1, 1)
    h_ = jnp.arange(CH // seq_per_w, dtype=jnp.int32).reshape(1, 1, -1, 1)
    s_ = jnp.arange(seq_per_w, dtype=jnp.int32).reshape(1, 1, 1, seq_per_w)
    sidx = ((w_ * seq_per_w + s_) * S + (CH // seq_per_w) * c_ + h_)
    sidx = sidx.reshape(NW, n_chunks, CH)
    out = _sc_lookup_ln(n_chunks, n_tok)(gidx, sidx, fused, pe[:SEQ],
                                         gamma, beta)
    return out.reshape(B, S, D_MODEL)
